# baseline (speedup 1.0000x reference)
import functools

import jax
import jax.numpy as jnp
from jax import lax
from jax.experimental import pallas as pl
from jax.experimental.pallas import tpu as pltpu
from jax.experimental.pallas import tpu_sc as plsc

N = 10000
E = 320000
NPAD = 12288
BLK = 400

NC = 2
NS = 16
L = 16
NH = NPAD // NC
NW = NH // NS
GCH = 128
EBLK = E // 128
WBLK = 157


def _sc_body(ei_hbm, ef_hbm, mem_hbm,
             win_out, memcol_out, efw_out,
             ev, winner_v, shared, mbuf, wslice, eidx2, colidx, colbuf,
             membuf, eft_buf, efw_buf, sem, rsem):
    c = lax.axis_index("c")
    s = lax.axis_index("s")
    node_base = c * NH
    lanes = lax.iota(jnp.int32, L)
    neg1 = jnp.full((L,), -1, jnp.int32)
    dump = jnp.full((L,), NH, jnp.int32) + lanes

    b0 = s * (EBLK // NS) + jnp.minimum(s, 3)
    rows_cp = pltpu.async_copy(ei_hbm.at[pl.ds(b0 * 256, WBLK * 256)], ev,
                               rsem)

    def init_body(i, _):
        winner_v[pl.ds(i * L, L)] = neg1
        return 0
    lax.fori_loop(0, (NH + L) // L, init_body, 0)
    rows_cp.wait()

    with jax.named_scope("p1_scan"):
        def scan_body(b, val):
            for i in range(8):
                r = ev[pl.ds(b * 256 + i * L, L)]
                lidx = plsc.bitcast(r - node_base, jnp.uint32)
                idxc = plsc.bitcast(
                    jnp.minimum(lidx, plsc.bitcast(dump, jnp.uint32)),
                    jnp.int32)
                plsc.store_scatter(winner_v, [idxc], val + i * L)
            return val + 128
        lax.fori_loop(0, WBLK, scan_body, b0 * 128 + lanes)

    out_base = node_base + s * NW
    with jax.named_scope("p2_merge"):
        pltpu.sync_copy(winner_v.at[pl.ds(0, NH)], shared.at[s])
        plsc.subcore_barrier()
        pltpu.sync_copy(shared.at[:, pl.ds(s * NW, NW)], mbuf)

        def merge_body(k, _):
            acc = neg1
            for j in range(NS):
                acc = jnp.maximum(acc, mbuf[j, pl.ds(k * L, L)])
            wslice[pl.ds(k * L, L)] = acc
            dummy = out_base + k * L + lanes
            e = jnp.where(acc >= 0, acc, dummy)
            hi = lax.shift_right_logical(e, 7)
            colidx[pl.ds(k * L, L)] = (
                lax.shift_left(hi, 8) + (e & 127) + 128)
            for f in range(16):
                eidx2[f, pl.ds(k * L, L)] = e + f * E
            return 0
        lax.fori_loop(0, NW // L, merge_body, 0)
    win_cp = pltpu.async_copy(wslice, win_out.at[pl.ds(out_base, NW)], rsem)

    with jax.named_scope("p3_gather"):
        nch = NW // GCH
        col_cps = [
            pltpu.async_copy(ei_hbm.at[colidx.at[pl.ds(j * GCH, GCH)]],
                             colbuf.at[pl.ds(j * GCH, GCH)], sem)
            for j in range(nch)
        ]
        ef_cps = [
            pltpu.async_copy(ef_hbm.at[eidx2.at[f, pl.ds(j * GCH, GCH)]],
                             eft_buf.at[f, pl.ds(j * GCH, GCH)], sem)
            for f in range(16)
            for j in range(nch)
        ]
        for cp in col_cps:
            cp.wait()
        mem_cps = [
            pltpu.async_copy(mem_hbm.at[colbuf.at[pl.ds(j * GCH, GCH)]],
                             membuf.at[pl.ds(j * GCH, GCH)], sem)
            for j in range(nch)
        ]
        for cp in ef_cps:
            cp.wait()

        def tr_body(g, _):
            nvec = g * L + lanes
            for f in range(16):
                v = eft_buf[f, pl.ds(g * L, L)]
                plsc.store_scatter(efw_buf, [nvec, jnp.full((L,), f,
                                                            jnp.int32)], v)
            return 0
        lax.fori_loop(0, NW // L, tr_body, 0)
        pltpu.sync_copy(efw_buf, efw_out.at[pl.ds(out_base, NW)])
        for cp in mem_cps:
            cp.wait()
        pltpu.sync_copy(membuf, memcol_out.at[pl.ds(out_base, NW)])
    win_cp.wait()


_sc_gather = functools.partial(
    pl.kernel,
    out_type=[
        jax.ShapeDtypeStruct((NPAD,), jnp.int32),
        jax.ShapeDtypeStruct((NPAD, 128), jnp.float32),
        jax.ShapeDtypeStruct((NPAD, 16), jnp.float32),
    ],
    mesh=plsc.VectorSubcoreMesh(core_axis_name="c", subcore_axis_name="s"),
    scratch_types=[
        pltpu.VMEM((WBLK * 256,), jnp.int32),
        pltpu.VMEM((NH + L,), jnp.int32),
        pltpu.VMEM_SHARED((NS, NH), jnp.int32),
        pltpu.VMEM((NS, NW), jnp.int32),
        pltpu.VMEM((NW,), jnp.int32),
        pltpu.VMEM((16, NW), jnp.int32),
        pltpu.VMEM((NW,), jnp.int32),
        pltpu.VMEM((NW,), jnp.int32),
        pltpu.VMEM((NW, 128), jnp.float32),
        pltpu.VMEM((16, NW), jnp.float32),
        pltpu.VMEM((NW, 16), jnp.float32),
        pltpu.SemaphoreType.DMA,
        pltpu.SemaphoreType.DMA,
    ],
    compiler_params=pltpu.CompilerParams(needs_layout_passes=False,
                                         use_tc_tiling_on_sc=False),
)(_sc_body)


def _dense_body(mem_ref, memcol_ref, nf_ref, ef_ref, win_ref,
                w1a_ref, w1b_ref, w1c_ref, b1_ref, w2_ref, b2_ref,
                wih_ref, bih_ref, whh_ref, bhh_ref,
                wemba_ref, wembb_ref, bemb_ref,
                emb_out, mem_out):
    f32 = jnp.float32
    bf16 = jnp.bfloat16
    mem = mem_ref[...]
    memb = mem.astype(bf16)

    x1 = (jnp.dot(memb, w1a_ref[...], preferred_element_type=f32)
          + jnp.dot(memcol_ref[...].astype(bf16), w1b_ref[...],
                    preferred_element_type=f32)
          + jnp.dot(ef_ref[...].astype(bf16), w1c_ref[...],
                    preferred_element_type=f32)
          + b1_ref[...])
    h1 = jnp.maximum(x1, 0.0)
    msg = (jnp.dot(h1.astype(bf16), w2_ref[...], preferred_element_type=f32)
           + b2_ref[...])

    gi = (jnp.dot(msg.astype(bf16), wih_ref[...], preferred_element_type=f32)
          + bih_ref[...])
    gh = jnp.dot(memb, whh_ref[...], preferred_element_type=f32) + bhh_ref[...]
    r = jax.nn.sigmoid(gi[:, :128] + gh[:, :128])
    z = jax.nn.sigmoid(gi[:, 128:256] + gh[:, 128:256])
    n = jnp.tanh(gi[:, 256:] + r * gh[:, 256:])
    gru = (1.0 - z) * n + z * mem

    mask = win_ref[...] >= 0
    newmem = jnp.where(mask, gru, mem)

    emb = (jnp.dot(newmem.astype(bf16), wemba_ref[...],
                   preferred_element_type=f32)
           + jnp.dot(nf_ref[...].astype(bf16), wembb_ref[...],
                     preferred_element_type=f32)
           + bemb_ref[...])
    emb_out[...] = emb
    mem_out[...] = newmem


def kernel(node_features, edge_index, edge_features, memory,
           W1, b1, W2, b2, Wih, bih, Whh, bhh, Wemb, bemb):
    ei_blocked = edge_index.reshape(2, EBLK, 128).transpose(1, 0, 2).reshape(-1)
    ef_flat = edge_features.T.reshape(-1)
    win_p, memcol_p, efw_p = _sc_gather(ei_blocked, ef_flat, memory)
    win_p = win_p.reshape(NPAD, 1)

    grid = N // BLK
    row_spec = lambda w: pl.BlockSpec((BLK, w), lambda i: (i, 0))
    full_spec = lambda a, b: pl.BlockSpec((a, b), lambda i: (0, 0))

    emb, newmem = pl.pallas_call(
        _dense_body,
        grid=(grid,),
        in_specs=[
            row_spec(128), row_spec(128), row_spec(128), row_spec(16),
            row_spec(1),
            full_spec(128, 128), full_spec(128, 128), full_spec(16, 128),
            full_spec(1, 128),
            full_spec(128, 128), full_spec(1, 128),
            full_spec(128, 384), full_spec(1, 384),
            full_spec(128, 384), full_spec(1, 384),
            full_spec(128, 128), full_spec(128, 128), full_spec(1, 128),
        ],
        out_specs=[row_spec(128), row_spec(128)],
        out_shape=[
            jax.ShapeDtypeStruct((N, 128), jnp.float32),
            jax.ShapeDtypeStruct((N, 128), jnp.float32),
        ],
    )(
        memory, memcol_p, node_features, efw_p, win_p,
        W1[:, :128].T.astype(jnp.bfloat16), W1[:, 128:256].T.astype(jnp.bfloat16),
        W1[:, 256:].T.astype(jnp.bfloat16), b1.reshape(1, 128),
        W2.T.astype(jnp.bfloat16), b2.reshape(1, 128),
        Wih.T.astype(jnp.bfloat16), bih.reshape(1, 384),
        Whh.T.astype(jnp.bfloat16), bhh.reshape(1, 384),
        Wemb[:, :128].T.astype(jnp.bfloat16), Wemb[:, 128:].T.astype(jnp.bfloat16),
        bemb.reshape(1, 128),
    )
    return emb, newmem


# speedup vs baseline: 1.1067x; 1.1067x over previous
import functools

import jax
import jax.numpy as jnp
from jax import lax
from jax.experimental import pallas as pl
from jax.experimental.pallas import tpu as pltpu
from jax.experimental.pallas import tpu_sc as plsc

N = 10000
E = 320000
NPAD = 12288
BLK = 1000

NC = 2
NS = 16
L = 16
NH = NPAD // NC
NW = NH // NS
GCH = 128
EBLK = E // 128
WBLK = 157


def _sc_body(ei_hbm, ef_hbm, mem_hbm,
             win_out, memcol_out, efw_out,
             ev, winner_v, shared, mbuf, wslice, eidx2, colidx, colbuf,
             membuf, eft_buf, efw_buf, sem, rsem):
    c = lax.axis_index("c")
    s = lax.axis_index("s")
    node_base = c * NH
    lanes = lax.iota(jnp.int32, L)
    neg1 = jnp.full((L,), -1, jnp.int32)
    dump = jnp.full((L,), NH, jnp.int32) + lanes

    b0 = s * (EBLK // NS) + jnp.minimum(s, 3)
    rows_cp = pltpu.async_copy(ei_hbm.at[pl.ds(b0 * 256, WBLK * 256)], ev,
                               rsem)

    def init_body(i, _):
        winner_v[pl.ds(i * L, L)] = neg1
        return 0
    lax.fori_loop(0, (NH + L) // L, init_body, 0)
    rows_cp.wait()

    with jax.named_scope("p1_scan"):
        def scan_body(b, val):
            for i in range(8):
                r = ev[pl.ds(b * 256 + i * L, L)]
                lidx = plsc.bitcast(r - node_base, jnp.uint32)
                idxc = plsc.bitcast(
                    jnp.minimum(lidx, plsc.bitcast(dump, jnp.uint32)),
                    jnp.int32)
                plsc.store_scatter(winner_v, [idxc], val + i * L)
            return val + 128
        lax.fori_loop(0, WBLK, scan_body, b0 * 128 + lanes)

    out_base = node_base + s * NW
    with jax.named_scope("p2_merge"):
        pltpu.sync_copy(winner_v.at[pl.ds(0, NH)], shared.at[s])
        plsc.subcore_barrier()
        pltpu.sync_copy(shared.at[:, pl.ds(s * NW, NW)], mbuf)

        def merge_body(k, _):
            acc = neg1
            for j in range(NS):
                acc = jnp.maximum(acc, mbuf[j, pl.ds(k * L, L)])
            wslice[pl.ds(k * L, L)] = acc
            dummy = out_base + k * L + lanes
            e = jnp.where(acc >= 0, acc, dummy)
            hi = lax.shift_right_logical(e, 7)
            colidx[pl.ds(k * L, L)] = (
                lax.shift_left(hi, 8) + (e & 127) + 128)
            for f in range(16):
                eidx2[f, pl.ds(k * L, L)] = e + f * E
            return 0
        lax.fori_loop(0, NW // L, merge_body, 0)
    win_cp = pltpu.async_copy(wslice, win_out.at[pl.ds(out_base, NW)], rsem)

    with jax.named_scope("p3_gather"):
        nch = NW // GCH
        col_cps = [
            pltpu.async_copy(ei_hbm.at[colidx.at[pl.ds(j * GCH, GCH)]],
                             colbuf.at[pl.ds(j * GCH, GCH)], sem)
            for j in range(nch)
        ]
        ef_cps = [
            pltpu.async_copy(ef_hbm.at[eidx2.at[f, pl.ds(j * GCH, GCH)]],
                             eft_buf.at[f, pl.ds(j * GCH, GCH)], sem)
            for f in range(16)
            for j in range(nch)
        ]
        for cp in col_cps:
            cp.wait()
        mem_cps = [
            pltpu.async_copy(mem_hbm.at[colbuf.at[pl.ds(j * GCH, GCH)]],
                             membuf.at[pl.ds(j * GCH, GCH)], sem)
            for j in range(nch)
        ]
        for cp in ef_cps:
            cp.wait()

        def tr_body(g, _):
            nvec = g * L + lanes
            for f in range(16):
                v = eft_buf[f, pl.ds(g * L, L)]
                plsc.store_scatter(efw_buf, [nvec, jnp.full((L,), f,
                                                            jnp.int32)], v)
            return 0
        lax.fori_loop(0, NW // L, tr_body, 0)
        pltpu.sync_copy(efw_buf, efw_out.at[pl.ds(out_base, NW)])
        for cp in mem_cps:
            cp.wait()
        pltpu.sync_copy(membuf, memcol_out.at[pl.ds(out_base, NW)])
    win_cp.wait()


_sc_gather = functools.partial(
    pl.kernel,
    out_type=[
        jax.ShapeDtypeStruct((NPAD,), jnp.int32),
        jax.ShapeDtypeStruct((NPAD, 128), jnp.float32),
        jax.ShapeDtypeStruct((NPAD, 16), jnp.float32),
    ],
    mesh=plsc.VectorSubcoreMesh(core_axis_name="c", subcore_axis_name="s"),
    scratch_types=[
        pltpu.VMEM((WBLK * 256,), jnp.int32),
        pltpu.VMEM((NH + L,), jnp.int32),
        pltpu.VMEM_SHARED((NS, NH), jnp.int32),
        pltpu.VMEM((NS, NW), jnp.int32),
        pltpu.VMEM((NW,), jnp.int32),
        pltpu.VMEM((16, NW), jnp.int32),
        pltpu.VMEM((NW,), jnp.int32),
        pltpu.VMEM((NW,), jnp.int32),
        pltpu.VMEM((NW, 128), jnp.float32),
        pltpu.VMEM((16, NW), jnp.float32),
        pltpu.VMEM((NW, 16), jnp.float32),
        pltpu.SemaphoreType.DMA,
        pltpu.SemaphoreType.DMA,
    ],
    compiler_params=pltpu.CompilerParams(needs_layout_passes=False,
                                         use_tc_tiling_on_sc=False),
)(_sc_body)


def _dense_body(mem_ref, memcol_ref, nf_ref, ef_ref, win_ref,
                w1a_ref, w1b_ref, w1c_ref, b1_ref, w2_ref, b2_ref,
                wih_ref, bih_ref, whh_ref, bhh_ref,
                wemba_ref, wembb_ref, bemb_ref,
                emb_out, mem_out):
    f32 = jnp.float32
    mem = mem_ref[...]

    x1 = (jnp.dot(mem, w1a_ref[...], preferred_element_type=f32)
          + jnp.dot(memcol_ref[...], w1b_ref[...], preferred_element_type=f32)
          + jnp.dot(ef_ref[...], w1c_ref[...], preferred_element_type=f32)
          + b1_ref[...])
    h1 = jnp.maximum(x1, 0.0)
    msg = jnp.dot(h1, w2_ref[...], preferred_element_type=f32) + b2_ref[...]

    gi = jnp.dot(msg, wih_ref[...], preferred_element_type=f32) + bih_ref[...]
    gh = jnp.dot(mem, whh_ref[...], preferred_element_type=f32) + bhh_ref[...]
    r = jax.nn.sigmoid(gi[:, :128] + gh[:, :128])
    z = jax.nn.sigmoid(gi[:, 128:256] + gh[:, 128:256])
    n = jnp.tanh(gi[:, 256:] + r * gh[:, 256:])
    gru = (1.0 - z) * n + z * mem

    mask = win_ref[...] >= 0
    newmem = jnp.where(mask, gru, mem)

    emb = (jnp.dot(newmem, wemba_ref[...], preferred_element_type=f32)
           + jnp.dot(nf_ref[...], wembb_ref[...], preferred_element_type=f32)
           + bemb_ref[...])
    emb_out[...] = emb
    mem_out[...] = newmem


def kernel(node_features, edge_index, edge_features, memory,
           W1, b1, W2, b2, Wih, bih, Whh, bhh, Wemb, bemb):
    ei_blocked = edge_index.reshape(2, EBLK, 128).transpose(1, 0, 2).reshape(-1)
    ef_flat = edge_features.T.reshape(-1)
    win_p, memcol_p, efw_p = _sc_gather(ei_blocked, ef_flat, memory)
    win_p = win_p.reshape(NPAD, 1)

    grid = N // BLK
    row_spec = lambda w: pl.BlockSpec((BLK, w), lambda i: (i, 0))
    full_spec = lambda a, b: pl.BlockSpec((a, b), lambda i: (0, 0))

    emb, newmem = pl.pallas_call(
        _dense_body,
        grid=(grid,),
        in_specs=[
            row_spec(128), row_spec(128), row_spec(128), row_spec(16),
            row_spec(1),
            full_spec(128, 128), full_spec(128, 128), full_spec(16, 128),
            full_spec(1, 128),
            full_spec(128, 128), full_spec(1, 128),
            full_spec(128, 384), full_spec(1, 384),
            full_spec(128, 384), full_spec(1, 384),
            full_spec(128, 128), full_spec(128, 128), full_spec(1, 128),
        ],
        out_specs=[row_spec(128), row_spec(128)],
        out_shape=[
            jax.ShapeDtypeStruct((N, 128), jnp.float32),
            jax.ShapeDtypeStruct((N, 128), jnp.float32),
        ],
    )(
        memory, memcol_p, node_features, efw_p, win_p,
        W1[:, :128].T, W1[:, 128:256].T, W1[:, 256:].T, b1.reshape(1, 128),
        W2.T, b2.reshape(1, 128),
        Wih.T, bih.reshape(1, 384),
        Whh.T, bhh.reshape(1, 384),
        Wemb[:, :128].T, Wemb[:, 128:].T, bemb.reshape(1, 128),
    )
    return emb, newmem
